# Initial kernel scaffold; baseline (speedup 1.0000x reference)
import jax
import jax.numpy as jnp
from jax.experimental import pallas as pl


def kernel(z, edge_index, edge_weight, edge_attr, batch, emb_table, lin_emb_W, lin_emb_b, filt_W1, filt_b1, filt_W2, filt_b2, lin1_W, lin2_W, lin2_b, ro_W1, ro_b1, ro_W2, ro_b2):
    raise NotImplementedError("write your pallas kernel here")


# speedup vs baseline: 1.6583x; 1.6583x over previous
import functools
import math

import jax
import jax.numpy as jnp
from jax import lax
from jax.experimental import pallas as pl
from jax.experimental.pallas import tpu as pltpu
from jax.experimental.pallas import tpu_sc as plsc

N = 10000
E = 320000
C = 128
DE = 16
NE = 120
NG = 64
CUTOFF = 10.0

NC, NS = 2, 16
NW = NC * NS
EB = 128
NCHUNK = E // EB
BASE_CHUNKS = NCHUNK // NW
EXTRA_TILES = NCHUNK - BASE_CHUNKS * NW

NB = 1000
NBLK = N // NB
BE = 2560
EBLK = E // BE

_LOG2 = math.log(2.0)


def _ssp(v):
    return jnp.logaddexp(v, 0.0) - _LOG2


def _node_feats_body(z_ref, emb_ref, wemb_ref, bemb_ref, w1_ref, x_ref, th_ref):
    th = jnp.dot(emb_ref[...], wemb_ref[...],
                 preferred_element_type=jnp.float32) + bemb_ref[...]
    tx = jnp.dot(th, w1_ref[...], preferred_element_type=jnp.float32)
    zb = z_ref[0, 0, :]
    oh = (zb[:, None] == lax.broadcasted_iota(jnp.int32, (NB, NE), 1)
          ).astype(jnp.float32)
    x_ref[...] = jnp.dot(oh, tx, preferred_element_type=jnp.float32)
    th_ref[...] = th


def _node_feats(z3, emb_table, lin_emb_W, lin_emb_b, lin1_W):
    return pl.pallas_call(
        _node_feats_body,
        grid=(NBLK,),
        in_specs=[
            pl.BlockSpec((1, 1, NB), lambda i: (i, 0, 0)),
            pl.BlockSpec((NE, C), lambda i: (0, 0)),
            pl.BlockSpec((C, C), lambda i: (0, 0)),
            pl.BlockSpec((1, C), lambda i: (0, 0)),
            pl.BlockSpec((C, C), lambda i: (0, 0)),
        ],
        out_specs=[
            pl.BlockSpec((NB, C), lambda i: (i, 0)),
            pl.BlockSpec((NE, C), lambda i: (0, 0)),
        ],
        out_shape=[
            jax.ShapeDtypeStruct((N, C), jnp.float32),
            jax.ShapeDtypeStruct((NE, C), jnp.float32),
        ],
    )(z3, emb_table, lin_emb_W, lin_emb_b, lin1_W)


def _filter_body(ea_ref, ew_ref, w1_ref, b1_ref, w2_ref, b2_ref, wf_ref):
    a = _ssp(jnp.dot(ea_ref[...], w1_ref[...],
                     preferred_element_type=jnp.float32) + b1_ref[...])
    ccut = 0.5 * (jnp.cos(ew_ref[...] * (math.pi / CUTOFF)) + 1.0)
    wf_ref[...] = (jnp.dot(a, w2_ref[...],
                           preferred_element_type=jnp.float32)
                   + b2_ref[...]) * ccut


def _filter(edge_attr, ew_col, filt_W1, filt_b1, filt_W2, filt_b2):
    return pl.pallas_call(
        _filter_body,
        grid=(EBLK,),
        in_specs=[
            pl.BlockSpec((BE, DE), lambda i: (i, 0)),
            pl.BlockSpec((BE, 1), lambda i: (i, 0)),
            pl.BlockSpec((DE, C), lambda i: (0, 0)),
            pl.BlockSpec((1, C), lambda i: (0, 0)),
            pl.BlockSpec((C, C), lambda i: (0, 0)),
            pl.BlockSpec((1, C), lambda i: (0, 0)),
        ],
        out_specs=pl.BlockSpec((BE, C), lambda i: (i, 0)),
        out_shape=jax.ShapeDtypeStruct((E, C), jnp.float32),
    )(edge_attr, ew_col, filt_W1, filt_b1, filt_W2, filt_b2)


NPAD = 10240


def _sc_gather_mul_scatter(x, wf, src, dst, zeros):
    mesh = plsc.VectorSubcoreMesh(core_axis_name="c", subcore_axis_name="s")
    rows_per_sub = NPAD // NS

    @functools.partial(
        pl.kernel,
        out_type=jax.ShapeDtypeStruct((NC, NPAD, C), jnp.float32),
        mesh=mesh,
        scratch_types=[
            pltpu.VMEM((EB,), jnp.int32),
            pltpu.VMEM((EB,), jnp.int32),
            pltpu.VMEM((EB, C), jnp.float32),
            pltpu.VMEM((EB, C), jnp.float32),
            pltpu.VMEM_SHARED((NPAD, C), jnp.float32),
            pltpu.SemaphoreType.DMA,
            pltpu.SemaphoreType.DMA,
        ],
    )
    def k(x_hbm, wf_hbm, src_hbm, dst_hbm, zeros_hbm, out_hbm,
          sidx, didx, xrows, wfv, shared, sem1, sem2):
        c = lax.axis_index("c")
        s = lax.axis_index("s")
        wid = s * NC + c

        pltpu.sync_copy(zeros_hbm.at[pl.ds(s * rows_per_sub, rows_per_sub)],
                        shared.at[pl.ds(s * rows_per_sub, rows_per_sub)])
        plsc.subcore_barrier()

        nchunks = BASE_CHUNKS + jnp.where(wid < EXTRA_TILES, 1, 0)

        @pl.loop(0, nchunks)
        def _(kk):
            base = (kk * NW + wid) * EB
            pltpu.sync_copy(src_hbm.at[pl.ds(base, EB)], sidx)
            pltpu.sync_copy(dst_hbm.at[pl.ds(base, EB)], didx)
            cp_g = pltpu.async_copy(x_hbm.at[sidx], xrows, sem1)
            cp_w = pltpu.async_copy(wf_hbm.at[pl.ds(base, EB)], wfv, sem2)
            cp_g.wait()
            cp_w.wait()

            @pl.loop(0, EB)
            def _(r):
                for j in range(C // 16):
                    sl = (r, pl.ds(j * 16, 16))
                    xrows[sl] = xrows[sl] * wfv[sl]

            pltpu.sync_copy(xrows, shared.at[didx], add=True)

        plsc.subcore_barrier()
        pltpu.sync_copy(shared.at[pl.ds(s * rows_per_sub, rows_per_sub)],
                        out_hbm.at[c, pl.ds(s * rows_per_sub, rows_per_sub)])

    return k(x, wf, src, dst, zeros)


def _final_body(agg_ref, z_ref, b_ref, th_ref, w2_ref, b2_ref,
                row1_ref, rob1_ref, row2_ref, rob2_ref, out_ref, gacc):
    i = pl.program_id(0)

    @pl.when(i == 0)
    def _():
        gacc[...] = jnp.zeros_like(gacc)

    agg = agg_ref[0] + agg_ref[1]
    t = _ssp(jnp.dot(agg, w2_ref[...],
                     preferred_element_type=jnp.float32) + b2_ref[...])
    zb = z_ref[0, 0, :]
    oh = (zb[:, None] == lax.broadcasted_iota(jnp.int32, (NB, NE), 1)
          ).astype(jnp.float32)
    h = jnp.dot(oh, th_ref[...], preferred_element_type=jnp.float32)
    hn = h + t
    bb = b_ref[0, 0, :]
    sm = (lax.broadcasted_iota(jnp.int32, (NG, NB), 0) == bb[None, :]
          ).astype(jnp.float32)
    gacc[...] += jnp.dot(sm, hn, preferred_element_type=jnp.float32)

    r = _ssp(jnp.dot(gacc[...], row1_ref[...],
                     preferred_element_type=jnp.float32) + rob1_ref[...])
    fin = _ssp(jnp.sum(r * row2_ref[...], axis=1, keepdims=True)
               + rob2_ref[...])
    out_ref[...] = jnp.broadcast_to(fin, (NG, C))


def _final(agg2, z3, batch3, th, lin2_W, lin2_b, ro_W1, ro_b1, ro_w2row,
           ro_b2s):
    return pl.pallas_call(
        _final_body,
        grid=(NBLK,),
        in_specs=[
            pl.BlockSpec((NC, NB, C), lambda i: (0, i, 0)),
            pl.BlockSpec((1, 1, NB), lambda i: (i, 0, 0)),
            pl.BlockSpec((1, 1, NB), lambda i: (i, 0, 0)),
            pl.BlockSpec((NE, C), lambda i: (0, 0)),
            pl.BlockSpec((C, C), lambda i: (0, 0)),
            pl.BlockSpec((1, C), lambda i: (0, 0)),
            pl.BlockSpec((C, C), lambda i: (0, 0)),
            pl.BlockSpec((1, C), lambda i: (0, 0)),
            pl.BlockSpec((1, C), lambda i: (0, 0)),
            pl.BlockSpec((1, 1), lambda i: (0, 0)),
        ],
        out_specs=pl.BlockSpec((NG, C), lambda i: (0, 0)),
        out_shape=jax.ShapeDtypeStruct((NG, C), jnp.float32),
        scratch_shapes=[pltpu.VMEM((NG, C), jnp.float32)],
    )(agg2, z3, batch3, th, lin2_W, lin2_b, ro_W1, ro_b1, ro_w2row, ro_b2s)


def kernel(z, edge_index, edge_weight, edge_attr, batch,
           emb_table, lin_emb_W, lin_emb_b,
           filt_W1, filt_b1, filt_W2, filt_b2,
           lin1_W, lin2_W, lin2_b,
           ro_W1, ro_b1, ro_W2, ro_b2):
    src = edge_index[0]
    dst = edge_index[1]
    z3 = z.reshape(NBLK, 1, NB)
    batch3 = batch.reshape(NBLK, 1, NB)
    ew_col = edge_weight.reshape(E, 1)
    zeros = jnp.zeros((NPAD, C), jnp.float32)

    x, th = _node_feats(z3, emb_table, lin_emb_W,
                        lin_emb_b.reshape(1, C), lin1_W)
    wf = _filter(edge_attr, ew_col, filt_W1, filt_b1.reshape(1, C),
                 filt_W2, filt_b2.reshape(1, C))
    agg2 = _sc_gather_mul_scatter(x, wf, src, dst, zeros)[:, :N, :]
    out128 = _final(agg2, z3, batch3, th, lin2_W, lin2_b.reshape(1, C),
                    ro_W1, ro_b1.reshape(1, C), ro_W2.reshape(1, C),
                    ro_b2.reshape(1, 1))
    return out128[:, :1]


# speedup vs baseline: 2.1922x; 1.3220x over previous
import functools
import math

import jax
import jax.numpy as jnp
from jax import lax
from jax.experimental import pallas as pl
from jax.experimental.pallas import tpu as pltpu
from jax.experimental.pallas import tpu_sc as plsc

N = 10000
E = 320000
C = 128
DE = 16
NE = 120
NG = 64
CUTOFF = 10.0

NC, NS = 2, 16
NW = NC * NS
EB = 128
NCHUNK = E // EB
BASE_CHUNKS = NCHUNK // NW
EXTRA_TILES = NCHUNK - BASE_CHUNKS * NW

NB = 1000
NBLK = N // NB
BE = 2560
EBLK = E // BE

_LOG2 = math.log(2.0)


def _ssp(v):
    return jnp.maximum(v, 0.0) + jnp.log(1.0 + jnp.exp(-jnp.abs(v))) - _LOG2


_EXPM2_C = (1.00000007549535, -0.6931472067106202, 0.24022107355831787,
            -0.05550327214211269, 0.009676037097836711,
            -0.0013400432164353806)
_LOG2P_C = (4.886358038598699e-08, 1.4426867778259662, -0.7211146144033356,
            0.47832354486716927, -0.3459960124305023, 0.23923166296547654,
            -0.1345342541895199, 0.05027750736438183, -0.008874696649584096)
_LOG2E = 1.4426950408889634
_CCUT_C = (0.9999999945295116, -0.24999994550535493, 0.020833244608692517,
           -0.0006943901799044242, 1.2384941778828043e-05,
           -1.3539515410903824e-07, 8.622545874016637e-10)
_MAGIC = 12582912.0
_MAGIC_BITS = 0x4B400000
_ONE_BITS = 0x3F800000


def _poly(c, x):
    r = jnp.full_like(x, c[-1])
    for v in c[-2::-1]:
        r = r * x + v
    return r


def _ssp_fast(v):
    t = jnp.minimum(jnp.abs(v), 20.0)
    z = t * _LOG2E
    zb = z + _MAGIC
    ki = lax.bitcast_convert_type(zb, jnp.int32) - _MAGIC_BITS
    r = z - ki.astype(jnp.float32)
    p = _poly(_EXPM2_C, r)
    scale = lax.bitcast_convert_type(_ONE_BITS - (ki << 23), jnp.float32)
    u = scale * p
    l2 = _poly(_LOG2P_C, u)
    return jnp.maximum(v, 0.0) + l2 * _LOG2 - _LOG2


def _node_feats_body(z_ref, emb_ref, wemb_ref, bemb_ref, w1_ref, x_ref, th_ref):
    th = jnp.dot(emb_ref[...], wemb_ref[...],
                 preferred_element_type=jnp.float32) + bemb_ref[...]
    tx = jnp.dot(th, w1_ref[...], preferred_element_type=jnp.float32)
    zb = z_ref[0, 0, :]
    oh = (zb[:, None] == lax.broadcasted_iota(jnp.int32, (NB, NE), 1)
          ).astype(jnp.float32)
    x_ref[...] = jnp.dot(oh, tx, preferred_element_type=jnp.float32)
    th_ref[...] = th


def _node_feats(z3, emb_table, lin_emb_W, lin_emb_b, lin1_W):
    return pl.pallas_call(
        _node_feats_body,
        grid=(NBLK,),
        in_specs=[
            pl.BlockSpec((1, 1, NB), lambda i: (i, 0, 0)),
            pl.BlockSpec((NE, C), lambda i: (0, 0)),
            pl.BlockSpec((C, C), lambda i: (0, 0)),
            pl.BlockSpec((1, C), lambda i: (0, 0)),
            pl.BlockSpec((C, C), lambda i: (0, 0)),
        ],
        out_specs=[
            pl.BlockSpec((NB, C), lambda i: (i, 0)),
            pl.BlockSpec((NE, C), lambda i: (0, 0)),
        ],
        out_shape=[
            jax.ShapeDtypeStruct((N, C), jnp.float32),
            jax.ShapeDtypeStruct((NE, C), jnp.float32),
        ],
    )(z3, emb_table, lin_emb_W, lin_emb_b, lin1_W)


def _filter_body(ea_ref, ew_ref, w1_ref, b1_ref, w2_ref, b2_ref, wf_ref):
    a = _ssp_fast(jnp.dot(ea_ref[...].astype(jnp.bfloat16),
                          w1_ref[...].astype(jnp.bfloat16),
                          preferred_element_type=jnp.float32) + b1_ref[...])
    th = jnp.clip(ew_ref[...], 0.0, CUTOFF) * (math.pi / CUTOFF)
    ccut = _poly(_CCUT_C, th * th)
    wf_ref[...] = (jnp.dot(a.astype(jnp.bfloat16),
                           w2_ref[...].astype(jnp.bfloat16),
                           preferred_element_type=jnp.float32)
                   + b2_ref[...]) * ccut


def _filter(edge_attr, ew_col, filt_W1, filt_b1, filt_W2, filt_b2):
    return pl.pallas_call(
        _filter_body,
        grid=(EBLK,),
        in_specs=[
            pl.BlockSpec((BE, DE), lambda i: (i, 0)),
            pl.BlockSpec((BE, 1), lambda i: (i, 0)),
            pl.BlockSpec((DE, C), lambda i: (0, 0)),
            pl.BlockSpec((1, C), lambda i: (0, 0)),
            pl.BlockSpec((C, C), lambda i: (0, 0)),
            pl.BlockSpec((1, C), lambda i: (0, 0)),
        ],
        out_specs=pl.BlockSpec((BE, C), lambda i: (i, 0)),
        out_shape=jax.ShapeDtypeStruct((E, C), jnp.float32),
        compiler_params=pltpu.CompilerParams(
            dimension_semantics=("parallel",)),
    )(edge_attr, ew_col, filt_W1, filt_b1, filt_W2, filt_b2)


NPAD = 10240


def _sc_gather_mul_scatter(x, wf, src, dst, zeros):
    mesh = plsc.VectorSubcoreMesh(core_axis_name="c", subcore_axis_name="s")
    rows_per_sub = NPAD // NS

    @functools.partial(
        pl.kernel,
        out_type=jax.ShapeDtypeStruct((NC, NPAD, C), jnp.float32),
        mesh=mesh,
        scratch_types=[
            pltpu.VMEM((EB,), jnp.int32),
            pltpu.VMEM((EB,), jnp.int32),
            pltpu.VMEM((EB, C), jnp.float32),
            pltpu.VMEM((EB, C), jnp.float32),
            pltpu.VMEM_SHARED((NPAD, C), jnp.float32),
            pltpu.SemaphoreType.DMA,
            pltpu.SemaphoreType.DMA,
        ],
    )
    def k(x_hbm, wf_hbm, src_hbm, dst_hbm, zeros_hbm, out_hbm,
          sidx, didx, xrows, wfv, shared, sem1, sem2):
        c = lax.axis_index("c")
        s = lax.axis_index("s")
        wid = s * NC + c

        pltpu.sync_copy(zeros_hbm.at[pl.ds(s * rows_per_sub, rows_per_sub)],
                        shared.at[pl.ds(s * rows_per_sub, rows_per_sub)])
        plsc.subcore_barrier()

        nchunks = BASE_CHUNKS + jnp.where(wid < EXTRA_TILES, 1, 0)

        @pl.loop(0, nchunks)
        def _(kk):
            base = (kk * NW + wid) * EB
            pltpu.sync_copy(src_hbm.at[pl.ds(base, EB)], sidx)
            pltpu.sync_copy(dst_hbm.at[pl.ds(base, EB)], didx)
            cp_g = pltpu.async_copy(x_hbm.at[sidx], xrows, sem1)
            cp_w = pltpu.async_copy(wf_hbm.at[pl.ds(base, EB)], wfv, sem2)
            cp_g.wait()
            cp_w.wait()

            @pl.loop(0, EB)
            def _(r):
                for j in range(C // 16):
                    sl = (r, pl.ds(j * 16, 16))
                    xrows[sl] = xrows[sl] * wfv[sl]

            pltpu.sync_copy(xrows, shared.at[didx], add=True)

        plsc.subcore_barrier()
        pltpu.sync_copy(shared.at[pl.ds(s * rows_per_sub, rows_per_sub)],
                        out_hbm.at[c, pl.ds(s * rows_per_sub, rows_per_sub)])

    return k(x, wf, src, dst, zeros)


def _final_body(agg_ref, z_ref, b_ref, th_ref, w2_ref, b2_ref,
                row1_ref, rob1_ref, row2_ref, rob2_ref, out_ref, gacc):
    i = pl.program_id(0)

    @pl.when(i == 0)
    def _():
        gacc[...] = jnp.zeros_like(gacc)

    agg = agg_ref[0] + agg_ref[1]
    t = _ssp(jnp.dot(agg, w2_ref[...],
                     preferred_element_type=jnp.float32) + b2_ref[...])
    zb = z_ref[0, 0, :]
    oh = (zb[:, None] == lax.broadcasted_iota(jnp.int32, (NB, NE), 1)
          ).astype(jnp.float32)
    h = jnp.dot(oh, th_ref[...], preferred_element_type=jnp.float32)
    hn = h + t
    bb = b_ref[0, 0, :]
    sm = (lax.broadcasted_iota(jnp.int32, (NG, NB), 0) == bb[None, :]
          ).astype(jnp.float32)
    gacc[...] += jnp.dot(sm, hn, preferred_element_type=jnp.float32)

    r = _ssp(jnp.dot(gacc[...], row1_ref[...],
                     preferred_element_type=jnp.float32) + rob1_ref[...])
    fin = _ssp(jnp.sum(r * row2_ref[...], axis=1, keepdims=True)
               + rob2_ref[...])
    out_ref[...] = jnp.broadcast_to(fin, (NG, C))


def _final(agg2, z3, batch3, th, lin2_W, lin2_b, ro_W1, ro_b1, ro_w2row,
           ro_b2s):
    return pl.pallas_call(
        _final_body,
        grid=(NBLK,),
        in_specs=[
            pl.BlockSpec((NC, NB, C), lambda i: (0, i, 0)),
            pl.BlockSpec((1, 1, NB), lambda i: (i, 0, 0)),
            pl.BlockSpec((1, 1, NB), lambda i: (i, 0, 0)),
            pl.BlockSpec((NE, C), lambda i: (0, 0)),
            pl.BlockSpec((C, C), lambda i: (0, 0)),
            pl.BlockSpec((1, C), lambda i: (0, 0)),
            pl.BlockSpec((C, C), lambda i: (0, 0)),
            pl.BlockSpec((1, C), lambda i: (0, 0)),
            pl.BlockSpec((1, C), lambda i: (0, 0)),
            pl.BlockSpec((1, 1), lambda i: (0, 0)),
        ],
        out_specs=pl.BlockSpec((NG, C), lambda i: (0, 0)),
        out_shape=jax.ShapeDtypeStruct((NG, C), jnp.float32),
        scratch_shapes=[pltpu.VMEM((NG, C), jnp.float32)],
    )(agg2, z3, batch3, th, lin2_W, lin2_b, ro_W1, ro_b1, ro_w2row, ro_b2s)


def kernel(z, edge_index, edge_weight, edge_attr, batch,
           emb_table, lin_emb_W, lin_emb_b,
           filt_W1, filt_b1, filt_W2, filt_b2,
           lin1_W, lin2_W, lin2_b,
           ro_W1, ro_b1, ro_W2, ro_b2):
    src = edge_index[0]
    dst = edge_index[1]
    z3 = z.reshape(NBLK, 1, NB)
    batch3 = batch.reshape(NBLK, 1, NB)
    ew_col = edge_weight.reshape(E, 1)
    zeros = jnp.zeros((NPAD, C), jnp.float32)

    x, th = _node_feats(z3, emb_table, lin_emb_W,
                        lin_emb_b.reshape(1, C), lin1_W)
    wf = _filter(edge_attr, ew_col, filt_W1, filt_b1.reshape(1, C),
                 filt_W2, filt_b2.reshape(1, C))
    agg2 = _sc_gather_mul_scatter(x, wf, src, dst, zeros)[:, :N, :]
    out128 = _final(agg2, z3, batch3, th, lin2_W, lin2_b.reshape(1, C),
                    ro_W1, ro_b1.reshape(1, C), ro_W2.reshape(1, C),
                    ro_b2.reshape(1, 1))
    return out128[:, :1]


# speedup vs baseline: 2.2125x; 1.0093x over previous
import functools
import math

import jax
import jax.numpy as jnp
from jax import lax
from jax.experimental import pallas as pl
from jax.experimental.pallas import tpu as pltpu
from jax.experimental.pallas import tpu_sc as plsc

N = 10000
E = 320000
C = 128
DE = 16
NE = 120
NG = 64
CUTOFF = 10.0

NC, NS = 2, 16
NW = NC * NS
EB = 128
NCHUNK = E // EB
BASE_CHUNKS = NCHUNK // NW
EXTRA_TILES = NCHUNK - BASE_CHUNKS * NW

NB = 1000
NBLK = N // NB
BE = 2560
EBLK = E // BE

_LOG2 = math.log(2.0)


def _ssp(v):
    return jnp.maximum(v, 0.0) + jnp.log(1.0 + jnp.exp(-jnp.abs(v))) - _LOG2


_EXPM2_C = (1.00000007549535, -0.6931472067106202, 0.24022107355831787,
            -0.05550327214211269, 0.009676037097836711,
            -0.0013400432164353806)
_LOG2P_C = (4.886358038598699e-08, 1.4426867778259662, -0.7211146144033356,
            0.47832354486716927, -0.3459960124305023, 0.23923166296547654,
            -0.1345342541895199, 0.05027750736438183, -0.008874696649584096)
_LOG2E = 1.4426950408889634
_CCUT_C = (0.9999999945295116, -0.24999994550535493, 0.020833244608692517,
           -0.0006943901799044242, 1.2384941778828043e-05,
           -1.3539515410903824e-07, 8.622545874016637e-10)
_MAGIC = 12582912.0
_MAGIC_BITS = 0x4B400000
_ONE_BITS = 0x3F800000


def _poly(c, x):
    r = jnp.full_like(x, c[-1])
    for v in c[-2::-1]:
        r = r * x + v
    return r


def _ssp_fast(v):
    t = jnp.minimum(jnp.abs(v), 20.0)
    z = t * _LOG2E
    zb = z + _MAGIC
    ki = lax.bitcast_convert_type(zb, jnp.int32) - _MAGIC_BITS
    r = z - ki.astype(jnp.float32)
    p = _poly(_EXPM2_C, r)
    scale = lax.bitcast_convert_type(_ONE_BITS - (ki << 23), jnp.float32)
    u = scale * p
    l2 = _poly(_LOG2P_C, u)
    return jnp.maximum(v, 0.0) + l2 * _LOG2 - _LOG2


def _node_feats_body(z_ref, emb_ref, wemb_ref, bemb_ref, w1_ref, x_ref, th_ref):
    th = jnp.dot(emb_ref[...], wemb_ref[...],
                 preferred_element_type=jnp.float32) + bemb_ref[...]
    tx = jnp.dot(th, w1_ref[...], preferred_element_type=jnp.float32)
    zb = z_ref[0, 0, :]
    oh = (zb[:, None] == lax.broadcasted_iota(jnp.int32, (NB, NE), 1)
          ).astype(jnp.float32)
    x_ref[...] = jnp.dot(oh, tx, preferred_element_type=jnp.float32)
    th_ref[...] = th


def _node_feats(z3, emb_table, lin_emb_W, lin_emb_b, lin1_W):
    return pl.pallas_call(
        _node_feats_body,
        grid=(NBLK,),
        in_specs=[
            pl.BlockSpec((1, 1, NB), lambda i: (i, 0, 0)),
            pl.BlockSpec((NE, C), lambda i: (0, 0)),
            pl.BlockSpec((C, C), lambda i: (0, 0)),
            pl.BlockSpec((1, C), lambda i: (0, 0)),
            pl.BlockSpec((C, C), lambda i: (0, 0)),
        ],
        out_specs=[
            pl.BlockSpec((NB, C), lambda i: (i, 0)),
            pl.BlockSpec((NE, C), lambda i: (0, 0)),
        ],
        out_shape=[
            jax.ShapeDtypeStruct((N, C), jnp.float32),
            jax.ShapeDtypeStruct((NE, C), jnp.float32),
        ],
    )(z3, emb_table, lin_emb_W, lin_emb_b, lin1_W)


def _filter_body(ea_ref, ew_ref, w1_ref, b1_ref, w2_ref, b2_ref, wf_ref):
    a = _ssp_fast(jnp.dot(ea_ref[...], w1_ref[...],
                          preferred_element_type=jnp.float32) + b1_ref[...])
    th = jnp.clip(ew_ref[...], 0.0, CUTOFF) * (math.pi / CUTOFF)
    ccut = _poly(_CCUT_C, th * th)
    wf_ref[...] = (jnp.dot(a, w2_ref[...],
                           preferred_element_type=jnp.float32)
                   + b2_ref[...]) * ccut


def _filter(edge_attr, ew_col, filt_W1, filt_b1, filt_W2, filt_b2):
    return pl.pallas_call(
        _filter_body,
        grid=(EBLK,),
        in_specs=[
            pl.BlockSpec((BE, DE), lambda i: (i, 0)),
            pl.BlockSpec((BE, 1), lambda i: (i, 0)),
            pl.BlockSpec((DE, C), lambda i: (0, 0)),
            pl.BlockSpec((1, C), lambda i: (0, 0)),
            pl.BlockSpec((C, C), lambda i: (0, 0)),
            pl.BlockSpec((1, C), lambda i: (0, 0)),
        ],
        out_specs=pl.BlockSpec((BE, C), lambda i: (i, 0)),
        out_shape=jax.ShapeDtypeStruct((E, C), jnp.float32),
        compiler_params=pltpu.CompilerParams(
            dimension_semantics=("parallel",)),
    )(edge_attr, ew_col, filt_W1, filt_b1, filt_W2, filt_b2)


NPAD = 10240


def _sc_gather_mul_scatter(x, wf, src, dst, zeros):
    mesh = plsc.VectorSubcoreMesh(core_axis_name="c", subcore_axis_name="s")
    rows_per_sub = NPAD // NS

    @functools.partial(
        pl.kernel,
        out_type=jax.ShapeDtypeStruct((NC, NPAD, C), jnp.float32),
        mesh=mesh,
        scratch_types=[
            pltpu.VMEM((EB,), jnp.int32),
            pltpu.VMEM((EB,), jnp.int32),
            pltpu.VMEM((EB, C), jnp.float32),
            pltpu.VMEM((EB, C), jnp.float32),
            pltpu.VMEM_SHARED((NPAD, C), jnp.float32),
            pltpu.SemaphoreType.DMA,
            pltpu.SemaphoreType.DMA,
        ],
    )
    def k(x_hbm, wf_hbm, src_hbm, dst_hbm, zeros_hbm, out_hbm,
          sidx, didx, xrows, wfv, shared, sem1, sem2):
        c = lax.axis_index("c")
        s = lax.axis_index("s")
        wid = s * NC + c

        pltpu.sync_copy(zeros_hbm.at[pl.ds(s * rows_per_sub, rows_per_sub)],
                        shared.at[pl.ds(s * rows_per_sub, rows_per_sub)])
        plsc.subcore_barrier()

        nchunks = BASE_CHUNKS + jnp.where(wid < EXTRA_TILES, 1, 0)

        @pl.loop(0, nchunks)
        def _(kk):
            base = (kk * NW + wid) * EB
            pltpu.sync_copy(src_hbm.at[pl.ds(base, EB)], sidx)
            pltpu.sync_copy(dst_hbm.at[pl.ds(base, EB)], didx)
            cp_g = pltpu.async_copy(x_hbm.at[sidx], xrows, sem1)
            cp_w = pltpu.async_copy(wf_hbm.at[pl.ds(base, EB)], wfv, sem2)
            cp_g.wait()
            cp_w.wait()

            @pl.loop(0, EB)
            def _(r):
                for j in range(C // 16):
                    sl = (r, pl.ds(j * 16, 16))
                    xrows[sl] = xrows[sl] * wfv[sl]

            pltpu.sync_copy(xrows, shared.at[didx], add=True)

        plsc.subcore_barrier()
        pltpu.sync_copy(shared.at[pl.ds(s * rows_per_sub, rows_per_sub)],
                        out_hbm.at[c, pl.ds(s * rows_per_sub, rows_per_sub)])

    return k(x, wf, src, dst, zeros)


def _final_body(agg_ref, z_ref, b_ref, th_ref, w2_ref, b2_ref,
                row1_ref, rob1_ref, row2_ref, rob2_ref, out_ref, gacc):
    i = pl.program_id(0)

    @pl.when(i == 0)
    def _():
        gacc[...] = jnp.zeros_like(gacc)

    agg = agg_ref[0] + agg_ref[1]
    t = _ssp(jnp.dot(agg, w2_ref[...],
                     preferred_element_type=jnp.float32) + b2_ref[...])
    zb = z_ref[0, 0, :]
    oh = (zb[:, None] == lax.broadcasted_iota(jnp.int32, (NB, NE), 1)
          ).astype(jnp.float32)
    h = jnp.dot(oh, th_ref[...], preferred_element_type=jnp.float32)
    hn = h + t
    bb = b_ref[0, 0, :]
    sm = (lax.broadcasted_iota(jnp.int32, (NG, NB), 0) == bb[None, :]
          ).astype(jnp.float32)
    gacc[...] += jnp.dot(sm, hn, preferred_element_type=jnp.float32)

    r = _ssp(jnp.dot(gacc[...], row1_ref[...],
                     preferred_element_type=jnp.float32) + rob1_ref[...])
    fin = _ssp(jnp.sum(r * row2_ref[...], axis=1, keepdims=True)
               + rob2_ref[...])
    out_ref[...] = jnp.broadcast_to(fin, (NG, C))


def _final(agg2, z3, batch3, th, lin2_W, lin2_b, ro_W1, ro_b1, ro_w2row,
           ro_b2s):
    return pl.pallas_call(
        _final_body,
        grid=(NBLK,),
        in_specs=[
            pl.BlockSpec((NC, NB, C), lambda i: (0, i, 0)),
            pl.BlockSpec((1, 1, NB), lambda i: (i, 0, 0)),
            pl.BlockSpec((1, 1, NB), lambda i: (i, 0, 0)),
            pl.BlockSpec((NE, C), lambda i: (0, 0)),
            pl.BlockSpec((C, C), lambda i: (0, 0)),
            pl.BlockSpec((1, C), lambda i: (0, 0)),
            pl.BlockSpec((C, C), lambda i: (0, 0)),
            pl.BlockSpec((1, C), lambda i: (0, 0)),
            pl.BlockSpec((1, C), lambda i: (0, 0)),
            pl.BlockSpec((1, 1), lambda i: (0, 0)),
        ],
        out_specs=pl.BlockSpec((NG, C), lambda i: (0, 0)),
        out_shape=jax.ShapeDtypeStruct((NG, C), jnp.float32),
        scratch_shapes=[pltpu.VMEM((NG, C), jnp.float32)],
    )(agg2, z3, batch3, th, lin2_W, lin2_b, ro_W1, ro_b1, ro_w2row, ro_b2s)


def kernel(z, edge_index, edge_weight, edge_attr, batch,
           emb_table, lin_emb_W, lin_emb_b,
           filt_W1, filt_b1, filt_W2, filt_b2,
           lin1_W, lin2_W, lin2_b,
           ro_W1, ro_b1, ro_W2, ro_b2):
    src = edge_index[0]
    dst = edge_index[1]
    z3 = z.reshape(NBLK, 1, NB)
    batch3 = batch.reshape(NBLK, 1, NB)
    ew_col = edge_weight.reshape(E, 1)
    zeros = jnp.zeros((NPAD, C), jnp.float32)

    x, th = _node_feats(z3, emb_table, lin_emb_W,
                        lin_emb_b.reshape(1, C), lin1_W)
    wf = _filter(edge_attr, ew_col, filt_W1, filt_b1.reshape(1, C),
                 filt_W2, filt_b2.reshape(1, C))
    agg2 = _sc_gather_mul_scatter(x, wf, src, dst, zeros)[:, :N, :]
    out128 = _final(agg2, z3, batch3, th, lin2_W, lin2_b.reshape(1, C),
                    ro_W1, ro_b1.reshape(1, C), ro_W2.reshape(1, C),
                    ro_b2.reshape(1, 1))
    return out128[:, :1]


# speedup vs baseline: 2.8759x; 1.2999x over previous
import functools
import math

import jax
import jax.numpy as jnp
from jax import lax
from jax.experimental import pallas as pl
from jax.experimental.pallas import tpu as pltpu
from jax.experimental.pallas import tpu_sc as plsc

N = 10000
E = 320000
C = 128
DE = 16
NE = 120
NG = 64
CUTOFF = 10.0

NC, NS = 2, 16
NW = NC * NS
EB = 128
NCHUNK = E // EB
BASE_CHUNKS = NCHUNK // NW
EXTRA_TILES = NCHUNK - BASE_CHUNKS * NW

NB = 1000
NBLK = N // NB
BE = 2560
EBLK = E // BE

_LOG2 = math.log(2.0)


def _ssp(v):
    return jnp.maximum(v, 0.0) + jnp.log(1.0 + jnp.exp(-jnp.abs(v))) - _LOG2


_EXPM2_C = (1.00000007549535, -0.6931472067106202, 0.24022107355831787,
            -0.05550327214211269, 0.009676037097836711,
            -0.0013400432164353806)
_LOG2P_C = (4.886358038598699e-08, 1.4426867778259662, -0.7211146144033356,
            0.47832354486716927, -0.3459960124305023, 0.23923166296547654,
            -0.1345342541895199, 0.05027750736438183, -0.008874696649584096)
_LOG2E = 1.4426950408889634
_CCUT_C = (0.9999999945295116, -0.24999994550535493, 0.020833244608692517,
           -0.0006943901799044242, 1.2384941778828043e-05,
           -1.3539515410903824e-07, 8.622545874016637e-10)
_MAGIC = 12582912.0
_MAGIC_BITS = 0x4B400000
_ONE_BITS = 0x3F800000


def _poly(c, x):
    r = jnp.full_like(x, c[-1])
    for v in c[-2::-1]:
        r = r * x + v
    return r


def _ssp_fast(v):
    t = jnp.minimum(jnp.abs(v), 20.0)
    z = t * _LOG2E
    zb = z + _MAGIC
    ki = lax.bitcast_convert_type(zb, jnp.int32) - _MAGIC_BITS
    r = z - ki.astype(jnp.float32)
    p = _poly(_EXPM2_C, r)
    scale = lax.bitcast_convert_type(_ONE_BITS - (ki << 23), jnp.float32)
    u = scale * p
    l2 = _poly(_LOG2P_C, u)
    return jnp.maximum(v, 0.0) + l2 * _LOG2 - _LOG2


def _node_feats_body(z_ref, emb_ref, wemb_ref, bemb_ref, w1_ref, x_ref, th_ref):
    th = jnp.dot(emb_ref[...], wemb_ref[...],
                 preferred_element_type=jnp.float32) + bemb_ref[...]
    tx = jnp.dot(th, w1_ref[...], preferred_element_type=jnp.float32)
    zb = z_ref[0, 0, :]
    oh = (zb[:, None] == lax.broadcasted_iota(jnp.int32, (NB, NE), 1)
          ).astype(jnp.float32)
    x_ref[...] = jnp.dot(oh, tx, preferred_element_type=jnp.float32)
    th_ref[...] = th


def _node_feats(z3, emb_table, lin_emb_W, lin_emb_b, lin1_W):
    return pl.pallas_call(
        _node_feats_body,
        grid=(NBLK,),
        in_specs=[
            pl.BlockSpec((1, 1, NB), lambda i: (i, 0, 0)),
            pl.BlockSpec((NE, C), lambda i: (0, 0)),
            pl.BlockSpec((C, C), lambda i: (0, 0)),
            pl.BlockSpec((1, C), lambda i: (0, 0)),
            pl.BlockSpec((C, C), lambda i: (0, 0)),
        ],
        out_specs=[
            pl.BlockSpec((NB, C), lambda i: (i, 0)),
            pl.BlockSpec((NE, C), lambda i: (0, 0)),
        ],
        out_shape=[
            jax.ShapeDtypeStruct((N, C), jnp.float32),
            jax.ShapeDtypeStruct((NE, C), jnp.float32),
        ],
    )(z3, emb_table, lin_emb_W, lin_emb_b, lin1_W)


_EROW = BE // 128


def _filter_body(eat_ref, ew_ref, w1_ref, b1_ref, w2_ref, b2_ref, wf_ref):
    a = _ssp_fast(lax.dot_general(eat_ref[...], w1_ref[...],
                                  (((0,), (0,)), ((), ())),
                                  preferred_element_type=jnp.float32)
                  + b1_ref[...])
    th = jnp.clip(ew_ref[0], 0.0, CUTOFF) * (math.pi / CUTOFF)
    ccp = _poly(_CCUT_C, th * th)
    rsel = (lax.broadcasted_iota(jnp.int32, (BE, _EROW), 0) >> 7
            ) == lax.broadcasted_iota(jnp.int32, (BE, _EROW), 1)
    g = jnp.dot(rsel.astype(jnp.float32), ccp,
                preferred_element_type=jnp.float32)
    lsel = (lax.broadcasted_iota(jnp.int32, (BE, C), 0) & 127
            ) == lax.broadcasted_iota(jnp.int32, (BE, C), 1)
    ccut = jnp.sum(jnp.where(lsel, g, 0.0), axis=1, keepdims=True)
    wf_ref[...] = (jnp.dot(a, w2_ref[...],
                           preferred_element_type=jnp.float32)
                   + b2_ref[...]) * ccut


def _filter(ea_t, ew3, filt_W1, filt_b1, filt_W2, filt_b2):
    return pl.pallas_call(
        _filter_body,
        grid=(EBLK,),
        in_specs=[
            pl.BlockSpec((DE, BE), lambda i: (0, i)),
            pl.BlockSpec((1, _EROW, 128), lambda i: (i, 0, 0)),
            pl.BlockSpec((DE, C), lambda i: (0, 0)),
            pl.BlockSpec((1, C), lambda i: (0, 0)),
            pl.BlockSpec((C, C), lambda i: (0, 0)),
            pl.BlockSpec((1, C), lambda i: (0, 0)),
        ],
        out_specs=pl.BlockSpec((BE, C), lambda i: (i, 0)),
        out_shape=jax.ShapeDtypeStruct((E, C), jnp.float32),
    )(ea_t, ew3, filt_W1, filt_b1, filt_W2, filt_b2)


NPAD = 10240


def _sc_gather_mul_scatter(x, wf, src, dst, zeros):
    mesh = plsc.VectorSubcoreMesh(core_axis_name="c", subcore_axis_name="s")
    rows_per_sub = NPAD // NS

    @functools.partial(
        pl.kernel,
        out_type=jax.ShapeDtypeStruct((NC, NPAD, C), jnp.float32),
        mesh=mesh,
        scratch_types=[
            pltpu.VMEM((EB,), jnp.int32),
            pltpu.VMEM((EB,), jnp.int32),
            pltpu.VMEM((EB, C), jnp.float32),
            pltpu.VMEM((EB, C), jnp.float32),
            pltpu.VMEM_SHARED((NPAD, C), jnp.float32),
            pltpu.SemaphoreType.DMA,
            pltpu.SemaphoreType.DMA,
        ],
    )
    def k(x_hbm, wf_hbm, src_hbm, dst_hbm, zeros_hbm, out_hbm,
          sidx, didx, xrows, wfv, shared, sem1, sem2):
        c = lax.axis_index("c")
        s = lax.axis_index("s")
        wid = s * NC + c

        pltpu.sync_copy(zeros_hbm.at[pl.ds(s * rows_per_sub, rows_per_sub)],
                        shared.at[pl.ds(s * rows_per_sub, rows_per_sub)])
        plsc.subcore_barrier()

        nchunks = BASE_CHUNKS + jnp.where(wid < EXTRA_TILES, 1, 0)

        @pl.loop(0, nchunks)
        def _(kk):
            base = (kk * NW + wid) * EB
            pltpu.sync_copy(src_hbm.at[pl.ds(base, EB)], sidx)
            pltpu.sync_copy(dst_hbm.at[pl.ds(base, EB)], didx)
            cp_g = pltpu.async_copy(x_hbm.at[sidx], xrows, sem1)
            cp_w = pltpu.async_copy(wf_hbm.at[pl.ds(base, EB)], wfv, sem2)
            cp_g.wait()
            cp_w.wait()

            @pl.loop(0, EB)
            def _(r):
                for j in range(C // 16):
                    sl = (r, pl.ds(j * 16, 16))
                    xrows[sl] = xrows[sl] * wfv[sl]

            pltpu.sync_copy(xrows, shared.at[didx], add=True)

        plsc.subcore_barrier()
        pltpu.sync_copy(shared.at[pl.ds(s * rows_per_sub, rows_per_sub)],
                        out_hbm.at[c, pl.ds(s * rows_per_sub, rows_per_sub)])

    return k(x, wf, src, dst, zeros)


def _final_body(agg_ref, z_ref, b_ref, th_ref, w2_ref, b2_ref,
                row1_ref, rob1_ref, row2_ref, rob2_ref, out_ref, gacc):
    i = pl.program_id(0)

    @pl.when(i == 0)
    def _():
        gacc[...] = jnp.zeros_like(gacc)

    agg = agg_ref[0] + agg_ref[1]
    t = _ssp(jnp.dot(agg, w2_ref[...],
                     preferred_element_type=jnp.float32) + b2_ref[...])
    zb = z_ref[0, 0, :]
    oh = (zb[:, None] == lax.broadcasted_iota(jnp.int32, (NB, NE), 1)
          ).astype(jnp.float32)
    h = jnp.dot(oh, th_ref[...], preferred_element_type=jnp.float32)
    hn = h + t
    bb = b_ref[0, 0, :]
    sm = (lax.broadcasted_iota(jnp.int32, (NG, NB), 0) == bb[None, :]
          ).astype(jnp.float32)
    gacc[...] += jnp.dot(sm, hn, preferred_element_type=jnp.float32)

    r = _ssp(jnp.dot(gacc[...], row1_ref[...],
                     preferred_element_type=jnp.float32) + rob1_ref[...])
    fin = _ssp(jnp.sum(r * row2_ref[...], axis=1, keepdims=True)
               + rob2_ref[...])
    out_ref[...] = jnp.broadcast_to(fin, (NG, C))


def _final(agg2, z3, batch3, th, lin2_W, lin2_b, ro_W1, ro_b1, ro_w2row,
           ro_b2s):
    return pl.pallas_call(
        _final_body,
        grid=(NBLK,),
        in_specs=[
            pl.BlockSpec((NC, NB, C), lambda i: (0, i, 0)),
            pl.BlockSpec((1, 1, NB), lambda i: (i, 0, 0)),
            pl.BlockSpec((1, 1, NB), lambda i: (i, 0, 0)),
            pl.BlockSpec((NE, C), lambda i: (0, 0)),
            pl.BlockSpec((C, C), lambda i: (0, 0)),
            pl.BlockSpec((1, C), lambda i: (0, 0)),
            pl.BlockSpec((C, C), lambda i: (0, 0)),
            pl.BlockSpec((1, C), lambda i: (0, 0)),
            pl.BlockSpec((1, C), lambda i: (0, 0)),
            pl.BlockSpec((1, 1), lambda i: (0, 0)),
        ],
        out_specs=pl.BlockSpec((NG, C), lambda i: (0, 0)),
        out_shape=jax.ShapeDtypeStruct((NG, C), jnp.float32),
        scratch_shapes=[pltpu.VMEM((NG, C), jnp.float32)],
    )(agg2, z3, batch3, th, lin2_W, lin2_b, ro_W1, ro_b1, ro_w2row, ro_b2s)


def kernel(z, edge_index, edge_weight, edge_attr, batch,
           emb_table, lin_emb_W, lin_emb_b,
           filt_W1, filt_b1, filt_W2, filt_b2,
           lin1_W, lin2_W, lin2_b,
           ro_W1, ro_b1, ro_W2, ro_b2):
    src = edge_index[0]
    dst = edge_index[1]
    z3 = z.reshape(NBLK, 1, NB)
    batch3 = batch.reshape(NBLK, 1, NB)
    ea_t = edge_attr.T
    ew3 = edge_weight.reshape(EBLK, _EROW, 128)
    zeros = jnp.zeros((NPAD, C), jnp.float32)

    x, th = _node_feats(z3, emb_table, lin_emb_W,
                        lin_emb_b.reshape(1, C), lin1_W)
    wf = _filter(ea_t, ew3, filt_W1, filt_b1.reshape(1, C),
                 filt_W2, filt_b2.reshape(1, C))
    agg2 = _sc_gather_mul_scatter(x, wf, src, dst, zeros)[:, :N, :]
    out128 = _final(agg2, z3, batch3, th, lin2_W, lin2_b.reshape(1, C),
                    ro_W1, ro_b1.reshape(1, C), ro_W2.reshape(1, C),
                    ro_b2.reshape(1, 1))
    return out128[:, :1]


# speedup vs baseline: 3.6025x; 1.2526x over previous
import functools
import math

import jax
import jax.numpy as jnp
from jax import lax
from jax.experimental import pallas as pl
from jax.experimental.pallas import tpu as pltpu
from jax.experimental.pallas import tpu_sc as plsc

N = 10000
E = 320000
C = 128
DE = 16
NE = 120
NG = 64
CUTOFF = 10.0

NC, NS = 2, 16
NW = NC * NS
EB = 128
NCHUNK = E // EB
BASE_CHUNKS = NCHUNK // NW
EXTRA_TILES = NCHUNK - BASE_CHUNKS * NW

NB = 1000
NBLK = N // NB
BE = 2560
EBLK = E // BE

_LOG2 = math.log(2.0)


def _ssp(v):
    return jnp.maximum(v, 0.0) + jnp.log(1.0 + jnp.exp(-jnp.abs(v))) - _LOG2


_EXPM2_C = (1.00000007549535, -0.6931472067106202, 0.24022107355831787,
            -0.05550327214211269, 0.009676037097836711,
            -0.0013400432164353806)
_LOG2P_C = (4.886358038598699e-08, 1.4426867778259662, -0.7211146144033356,
            0.47832354486716927, -0.3459960124305023, 0.23923166296547654,
            -0.1345342541895199, 0.05027750736438183, -0.008874696649584096)
_LOG2E = 1.4426950408889634
_CCUT_C = (0.9999999945295116, -0.24999994550535493, 0.020833244608692517,
           -0.0006943901799044242, 1.2384941778828043e-05,
           -1.3539515410903824e-07, 8.622545874016637e-10)
_MAGIC = 12582912.0
_MAGIC_BITS = 0x4B400000
_ONE_BITS = 0x3F800000


def _poly(c, x):
    r = jnp.full_like(x, c[-1])
    for v in c[-2::-1]:
        r = r * x + v
    return r


def _ssp_fast(v):
    t = jnp.minimum(jnp.abs(v), 20.0)
    z = t * _LOG2E
    zb = z + _MAGIC
    ki = lax.bitcast_convert_type(zb, jnp.int32) - _MAGIC_BITS
    r = z - ki.astype(jnp.float32)
    p = _poly(_EXPM2_C, r)
    scale = lax.bitcast_convert_type(_ONE_BITS - (ki << 23), jnp.float32)
    u = scale * p
    l2 = _poly(_LOG2P_C, u)
    return jnp.maximum(v, 0.0) + l2 * _LOG2 - _LOG2


def _node_feats_body(z_ref, emb_ref, wemb_ref, bemb_ref, w1_ref, x_ref, th_ref):
    th = jnp.dot(emb_ref[...], wemb_ref[...],
                 preferred_element_type=jnp.float32) + bemb_ref[...]
    tx = jnp.dot(th, w1_ref[...], preferred_element_type=jnp.float32)
    zb = z_ref[0, 0, :]
    oh = (zb[:, None] == lax.broadcasted_iota(jnp.int32, (NB, NE), 1)
          ).astype(jnp.float32)
    x_ref[...] = jnp.dot(oh, tx, preferred_element_type=jnp.float32)
    th_ref[...] = th


def _node_feats(z3, emb_table, lin_emb_W, lin_emb_b, lin1_W):
    return pl.pallas_call(
        _node_feats_body,
        grid=(NBLK,),
        in_specs=[
            pl.BlockSpec((1, 1, NB), lambda i: (i, 0, 0)),
            pl.BlockSpec((NE, C), lambda i: (0, 0)),
            pl.BlockSpec((C, C), lambda i: (0, 0)),
            pl.BlockSpec((1, C), lambda i: (0, 0)),
            pl.BlockSpec((C, C), lambda i: (0, 0)),
        ],
        out_specs=[
            pl.BlockSpec((NB, C), lambda i: (i, 0)),
            pl.BlockSpec((NE, C), lambda i: (0, 0)),
        ],
        out_shape=[
            jax.ShapeDtypeStruct((N, C), jnp.float32),
            jax.ShapeDtypeStruct((NE, C), jnp.float32),
        ],
    )(z3, emb_table, lin_emb_W, lin_emb_b, lin1_W)


_EROW = BE // 128


def _filter_body(eat_ref, ew_ref, w1_ref, b1_ref, w2_ref, b2_ref, wf_ref):
    a = _ssp_fast(lax.dot_general(eat_ref[...], w1_ref[...],
                                  (((0,), (0,)), ((), ())),
                                  preferred_element_type=jnp.float32)
                  + b1_ref[...])
    th = jnp.clip(ew_ref[0], 0.0, CUTOFF) * (math.pi / CUTOFF)
    ccp = _poly(_CCUT_C, th * th)
    rsel = (lax.broadcasted_iota(jnp.int32, (BE, _EROW), 0) >> 7
            ) == lax.broadcasted_iota(jnp.int32, (BE, _EROW), 1)
    g = jnp.dot(rsel.astype(jnp.float32), ccp,
                preferred_element_type=jnp.float32)
    lsel = (lax.broadcasted_iota(jnp.int32, (BE, C), 0) & 127
            ) == lax.broadcasted_iota(jnp.int32, (BE, C), 1)
    ccut = jnp.sum(jnp.where(lsel, g, 0.0), axis=1, keepdims=True)
    wf_ref[...] = (jnp.dot(a, w2_ref[...],
                           preferred_element_type=jnp.float32)
                   + b2_ref[...]) * ccut


NSPLIT = 5
ESPLIT = E // NSPLIT
SBLK = EBLK // NSPLIT


def _filter(ea_t, ew3, filt_W1, filt_b1, filt_W2, filt_b2, split):
    off = split * SBLK
    return pl.pallas_call(
        _filter_body,
        grid=(SBLK,),
        in_specs=[
            pl.BlockSpec((DE, BE), lambda i: (0, i + off)),
            pl.BlockSpec((1, _EROW, 128), lambda i: (i + off, 0, 0)),
            pl.BlockSpec((DE, C), lambda i: (0, 0)),
            pl.BlockSpec((1, C), lambda i: (0, 0)),
            pl.BlockSpec((C, C), lambda i: (0, 0)),
            pl.BlockSpec((1, C), lambda i: (0, 0)),
        ],
        out_specs=pl.BlockSpec((BE, C), lambda i: (i, 0)),
        out_shape=jax.ShapeDtypeStruct((ESPLIT, C), jnp.float32),
    )(ea_t, ew3, filt_W1, filt_b1, filt_W2, filt_b2)


NPAD = 10240


def _sc_gather_mul_scatter(x, wf, src, dst, zeros, split):
    mesh = plsc.VectorSubcoreMesh(core_axis_name="c", subcore_axis_name="s")
    rows_per_sub = NPAD // NS
    nch_call = NCHUNK // NSPLIT
    base_chunks = nch_call // NW
    extra_tiles = nch_call - base_chunks * NW
    goff = split * nch_call * EB

    @functools.partial(
        pl.kernel,
        out_type=jax.ShapeDtypeStruct((NC, NPAD, C), jnp.float32),
        mesh=mesh,
        scratch_types=[
            pltpu.VMEM((EB,), jnp.int32),
            pltpu.VMEM((EB,), jnp.int32),
            pltpu.VMEM((EB, C), jnp.float32),
            pltpu.VMEM((EB, C), jnp.float32),
            pltpu.VMEM_SHARED((NPAD, C), jnp.float32),
            pltpu.SemaphoreType.DMA,
            pltpu.SemaphoreType.DMA,
        ],
    )
    def k(x_hbm, wf_hbm, src_hbm, dst_hbm, zeros_hbm, out_hbm,
          sidx, didx, xrows, wfv, shared, sem1, sem2):
        c = lax.axis_index("c")
        s = lax.axis_index("s")
        wid = s * NC + c

        pltpu.sync_copy(zeros_hbm.at[pl.ds(s * rows_per_sub, rows_per_sub)],
                        shared.at[pl.ds(s * rows_per_sub, rows_per_sub)])
        plsc.subcore_barrier()

        nchunks = base_chunks + jnp.where(wid < extra_tiles, 1, 0)

        @pl.loop(0, nchunks)
        def _(kk):
            base = (kk * NW + wid) * EB
            pltpu.sync_copy(src_hbm.at[pl.ds(goff + base, EB)], sidx)
            pltpu.sync_copy(dst_hbm.at[pl.ds(goff + base, EB)], didx)
            cp_g = pltpu.async_copy(x_hbm.at[sidx], xrows, sem1)
            cp_w = pltpu.async_copy(wf_hbm.at[pl.ds(base, EB)], wfv, sem2)
            cp_g.wait()
            cp_w.wait()

            @pl.loop(0, EB)
            def _(r):
                for j in range(C // 16):
                    sl = (r, pl.ds(j * 16, 16))
                    xrows[sl] = xrows[sl] * wfv[sl]

            pltpu.sync_copy(xrows, shared.at[didx], add=True)

        plsc.subcore_barrier()
        pltpu.sync_copy(shared.at[pl.ds(s * rows_per_sub, rows_per_sub)],
                        out_hbm.at[c, pl.ds(s * rows_per_sub, rows_per_sub)])

    return k(x, wf, src, dst, zeros)


def _final_body(*refs):
    agg_refs = refs[:NSPLIT]
    (z_ref, b_ref, th_ref, w2_ref, b2_ref,
     row1_ref, rob1_ref, row2_ref, rob2_ref, out_ref, gacc) = refs[NSPLIT:]
    i = pl.program_id(0)

    @pl.when(i == 0)
    def _():
        gacc[...] = jnp.zeros_like(gacc)

    agg = agg_refs[0][0] + agg_refs[0][1]
    for ar in agg_refs[1:]:
        agg = agg + ar[0] + ar[1]
    t = _ssp(jnp.dot(agg, w2_ref[...],
                     preferred_element_type=jnp.float32) + b2_ref[...])
    zb = z_ref[0, 0, :]
    oh = (zb[:, None] == lax.broadcasted_iota(jnp.int32, (NB, NE), 1)
          ).astype(jnp.float32)
    h = jnp.dot(oh, th_ref[...], preferred_element_type=jnp.float32)
    hn = h + t
    bb = b_ref[0, 0, :]
    sm = (lax.broadcasted_iota(jnp.int32, (NG, NB), 0) == bb[None, :]
          ).astype(jnp.float32)
    gacc[...] += jnp.dot(sm, hn, preferred_element_type=jnp.float32)

    r = _ssp(jnp.dot(gacc[...], row1_ref[...],
                     preferred_element_type=jnp.float32) + rob1_ref[...])
    fin = _ssp(jnp.sum(r * row2_ref[...], axis=1, keepdims=True)
               + rob2_ref[...])
    out_ref[...] = jnp.broadcast_to(fin, (NG, C))


def _final(aggs, z3, batch3, th, lin2_W, lin2_b, ro_W1, ro_b1, ro_w2row,
           ro_b2s):
    return pl.pallas_call(
        _final_body,
        grid=(NBLK,),
        in_specs=[
            pl.BlockSpec((NC, NB, C), lambda i: (0, i, 0))
            for _ in range(NSPLIT)
        ] + [
            pl.BlockSpec((1, 1, NB), lambda i: (i, 0, 0)),
            pl.BlockSpec((1, 1, NB), lambda i: (i, 0, 0)),
            pl.BlockSpec((NE, C), lambda i: (0, 0)),
            pl.BlockSpec((C, C), lambda i: (0, 0)),
            pl.BlockSpec((1, C), lambda i: (0, 0)),
            pl.BlockSpec((C, C), lambda i: (0, 0)),
            pl.BlockSpec((1, C), lambda i: (0, 0)),
            pl.BlockSpec((1, C), lambda i: (0, 0)),
            pl.BlockSpec((1, 1), lambda i: (0, 0)),
        ],
        out_specs=pl.BlockSpec((NG, C), lambda i: (0, 0)),
        out_shape=jax.ShapeDtypeStruct((NG, C), jnp.float32),
        scratch_shapes=[pltpu.VMEM((NG, C), jnp.float32)],
    )(*aggs, z3, batch3, th, lin2_W, lin2_b, ro_W1, ro_b1, ro_w2row, ro_b2s)


def kernel(z, edge_index, edge_weight, edge_attr, batch,
           emb_table, lin_emb_W, lin_emb_b,
           filt_W1, filt_b1, filt_W2, filt_b2,
           lin1_W, lin2_W, lin2_b,
           ro_W1, ro_b1, ro_W2, ro_b2):
    src = edge_index[0]
    dst = edge_index[1]
    z3 = z.reshape(NBLK, 1, NB)
    batch3 = batch.reshape(NBLK, 1, NB)
    ea_t = edge_attr.T
    ew3 = edge_weight.reshape(EBLK, _EROW, 128)
    zeros = jnp.zeros((NPAD, C), jnp.float32)

    x, th = _node_feats(z3, emb_table, lin_emb_W,
                        lin_emb_b.reshape(1, C), lin1_W)
    aggs = []
    for s in range(NSPLIT):
        wf = _filter(ea_t, ew3, filt_W1, filt_b1.reshape(1, C),
                     filt_W2, filt_b2.reshape(1, C), s)
        aggs.append(_sc_gather_mul_scatter(x, wf, src, dst, zeros, s))
    out128 = _final(aggs, z3, batch3, th, lin2_W, lin2_b.reshape(1, C),
                    ro_W1, ro_b1.reshape(1, C), ro_W2.reshape(1, C),
                    ro_b2.reshape(1, 1))
    return out128[:, :1]


# speedup vs baseline: 4.5423x; 1.2609x over previous
import functools
import math

import jax
import jax.numpy as jnp
from jax import lax
from jax.experimental import pallas as pl
from jax.experimental.pallas import tpu as pltpu
from jax.experimental.pallas import tpu_sc as plsc

N = 10000
E = 320000
C = 128
DE = 16
NE = 120
NG = 64
CUTOFF = 10.0

NC, NS = 2, 16
NW = NC * NS
EB = 64
NCHUNK = E // EB
BASE_CHUNKS = NCHUNK // NW
EXTRA_TILES = NCHUNK - BASE_CHUNKS * NW

NB = 1000
NBLK = N // NB
BE = 2560
EBLK = E // BE

_LOG2 = math.log(2.0)


def _ssp(v):
    return jnp.maximum(v, 0.0) + jnp.log(1.0 + jnp.exp(-jnp.abs(v))) - _LOG2


_EXPM2_C = (1.00000007549535, -0.6931472067106202, 0.24022107355831787,
            -0.05550327214211269, 0.009676037097836711,
            -0.0013400432164353806)
_LOG2P_C = (4.886358038598699e-08, 1.4426867778259662, -0.7211146144033356,
            0.47832354486716927, -0.3459960124305023, 0.23923166296547654,
            -0.1345342541895199, 0.05027750736438183, -0.008874696649584096)
_LOG2E = 1.4426950408889634
_CCUT_C = (0.9999999945295116, -0.24999994550535493, 0.020833244608692517,
           -0.0006943901799044242, 1.2384941778828043e-05,
           -1.3539515410903824e-07, 8.622545874016637e-10)
_MAGIC = 12582912.0
_MAGIC_BITS = 0x4B400000
_ONE_BITS = 0x3F800000


def _poly(c, x):
    r = jnp.full_like(x, c[-1])
    for v in c[-2::-1]:
        r = r * x + v
    return r


def _ssp_fast(v):
    t = jnp.minimum(jnp.abs(v), 20.0)
    z = t * _LOG2E
    zb = z + _MAGIC
    ki = lax.bitcast_convert_type(zb, jnp.int32) - _MAGIC_BITS
    r = z - ki.astype(jnp.float32)
    p = _poly(_EXPM2_C, r)
    scale = lax.bitcast_convert_type(_ONE_BITS - (ki << 23), jnp.float32)
    u = scale * p
    l2 = _poly(_LOG2P_C, u)
    return jnp.maximum(v, 0.0) + l2 * _LOG2 - _LOG2


def _node_feats_body(z_ref, emb_ref, wemb_ref, bemb_ref, w1_ref, x_ref, th_ref):
    th = jnp.dot(emb_ref[...], wemb_ref[...],
                 preferred_element_type=jnp.float32) + bemb_ref[...]
    tx = jnp.dot(th, w1_ref[...], preferred_element_type=jnp.float32)
    zb = z_ref[0, 0, :]
    oh = (zb[:, None] == lax.broadcasted_iota(jnp.int32, (NB, NE), 1)
          ).astype(jnp.float32)
    x_ref[...] = jnp.dot(oh, tx, preferred_element_type=jnp.float32)
    th_ref[...] = th


def _node_feats(z3, emb_table, lin_emb_W, lin_emb_b, lin1_W):
    return pl.pallas_call(
        _node_feats_body,
        grid=(NBLK,),
        in_specs=[
            pl.BlockSpec((1, 1, NB), lambda i: (i, 0, 0)),
            pl.BlockSpec((NE, C), lambda i: (0, 0)),
            pl.BlockSpec((C, C), lambda i: (0, 0)),
            pl.BlockSpec((1, C), lambda i: (0, 0)),
            pl.BlockSpec((C, C), lambda i: (0, 0)),
        ],
        out_specs=[
            pl.BlockSpec((NB, C), lambda i: (i, 0)),
            pl.BlockSpec((NE, C), lambda i: (0, 0)),
        ],
        out_shape=[
            jax.ShapeDtypeStruct((N, C), jnp.float32),
            jax.ShapeDtypeStruct((NE, C), jnp.float32),
        ],
    )(z3, emb_table, lin_emb_W, lin_emb_b, lin1_W)


_EROW = BE // 128


def _filter_body(eat_ref, ew_ref, w1_ref, b1_ref, w2_ref, b2_ref, wf_ref):
    a = _ssp_fast(lax.dot_general(eat_ref[...], w1_ref[...],
                                  (((0,), (0,)), ((), ())),
                                  preferred_element_type=jnp.float32)
                  + b1_ref[...])
    th = jnp.clip(ew_ref[0], 0.0, CUTOFF) * (math.pi / CUTOFF)
    ccp = _poly(_CCUT_C, th * th)
    rsel = (lax.broadcasted_iota(jnp.int32, (BE, _EROW), 0) >> 7
            ) == lax.broadcasted_iota(jnp.int32, (BE, _EROW), 1)
    g = jnp.dot(rsel.astype(jnp.float32), ccp,
                preferred_element_type=jnp.float32)
    lsel = (lax.broadcasted_iota(jnp.int32, (BE, C), 0) & 127
            ) == lax.broadcasted_iota(jnp.int32, (BE, C), 1)
    ccut = jnp.sum(jnp.where(lsel, g, 0.0), axis=1, keepdims=True)
    wf_ref[...] = (jnp.dot(a, w2_ref[...],
                           preferred_element_type=jnp.float32)
                   + b2_ref[...]) * ccut


NSPLIT = 5
ESPLIT = E // NSPLIT
SBLK = EBLK // NSPLIT


def _filter(ea_t, ew3, filt_W1, filt_b1, filt_W2, filt_b2, split):
    off = split * SBLK
    return pl.pallas_call(
        _filter_body,
        grid=(SBLK,),
        in_specs=[
            pl.BlockSpec((DE, BE), lambda i: (0, i + off)),
            pl.BlockSpec((1, _EROW, 128), lambda i: (i + off, 0, 0)),
            pl.BlockSpec((DE, C), lambda i: (0, 0)),
            pl.BlockSpec((1, C), lambda i: (0, 0)),
            pl.BlockSpec((C, C), lambda i: (0, 0)),
            pl.BlockSpec((1, C), lambda i: (0, 0)),
        ],
        out_specs=pl.BlockSpec((BE, C), lambda i: (i, 0)),
        out_shape=jax.ShapeDtypeStruct((ESPLIT, C), jnp.float32),
    )(ea_t, ew3, filt_W1, filt_b1, filt_W2, filt_b2)


NPAD = 10240


def _sc_gather_mul_scatter(x, wf, src, dst, zeros, split):
    mesh = plsc.VectorSubcoreMesh(core_axis_name="c", subcore_axis_name="s")
    rows_per_sub = NPAD // NS
    nch_call = NCHUNK // NSPLIT
    base_chunks = nch_call // NW
    extra_tiles = nch_call - base_chunks * NW
    goff = split * nch_call * EB

    maxn = (nch_call + NW - 1) // NW

    @functools.partial(
        pl.kernel,
        out_type=jax.ShapeDtypeStruct((NC, NPAD, C), jnp.float32),
        mesh=mesh,
        scratch_types=[
            pltpu.VMEM((EB,), jnp.int32),
            pltpu.VMEM((EB,), jnp.int32),
            pltpu.VMEM((EB,), jnp.int32),
            pltpu.VMEM((EB,), jnp.int32),
            pltpu.VMEM((EB, C), jnp.float32),
            pltpu.VMEM((EB, C), jnp.float32),
            pltpu.VMEM((EB, C), jnp.float32),
            pltpu.VMEM((EB, C), jnp.float32),
            pltpu.VMEM_SHARED((NPAD, C), jnp.float32),
            pltpu.SemaphoreType.DMA,
            pltpu.SemaphoreType.DMA,
            pltpu.SemaphoreType.DMA,
            pltpu.SemaphoreType.DMA,
            pltpu.SemaphoreType.DMA,
            pltpu.SemaphoreType.DMA,
        ],
    )
    def k(x_hbm, wf_hbm, src_hbm, dst_hbm, zeros_hbm, out_hbm,
          sidx0, sidx1, didx0, didx1, xrows0, xrows1, wfv0, wfv1, shared,
          isem0, isem1, gsem0, gsem1, wsem0, wsem1):
        c = lax.axis_index("c")
        s = lax.axis_index("s")
        wid = s * NC + c
        sidx = (sidx0, sidx1)
        didx = (didx0, didx1)
        xrows = (xrows0, xrows1)
        wfv = (wfv0, wfv1)
        isem = (isem0, isem1)
        gsem = (gsem0, gsem1)
        wsem = (wsem0, wsem1)

        pltpu.sync_copy(zeros_hbm.at[pl.ds(s * rows_per_sub, rows_per_sub)],
                        shared.at[pl.ds(s * rows_per_sub, rows_per_sub)])
        plsc.subcore_barrier()

        nchunks = base_chunks + jnp.where(wid < extra_tiles, 1, 0)

        def lbase(kk):
            return jnp.minimum(kk * NW + wid, nch_call - 1) * EB

        def issue_idx(kk, b):
            pltpu.async_copy(src_hbm.at[pl.ds(goff + lbase(kk), EB)],
                             sidx[b], isem[b])
            pltpu.async_copy(dst_hbm.at[pl.ds(goff + lbase(kk), EB)],
                             didx[b], isem[b])

        def wait_idx(b):
            pltpu.make_async_copy(src_hbm.at[pl.ds(0, EB)],
                                  sidx[b], isem[b]).wait()
            pltpu.make_async_copy(src_hbm.at[pl.ds(0, EB)],
                                  didx[b], isem[b]).wait()

        def issue_data(kk, b):
            pltpu.async_copy(x_hbm.at[sidx[b]], xrows[b], gsem[b])
            pltpu.async_copy(wf_hbm.at[pl.ds(lbase(kk), EB)], wfv[b], wsem[b])

        def wait_data(b):
            pltpu.make_async_copy(x_hbm.at[sidx[b]],
                                  xrows[b], gsem[b]).wait()
            pltpu.make_async_copy(wf_hbm.at[pl.ds(0, EB)],
                                  wfv[b], wsem[b]).wait()

        issue_idx(0, 0)
        wait_idx(0)
        issue_data(0, 0)
        issue_idx(1, 1)

        @pl.loop(0, maxn // 2)
        def _(kh):
            for b in (0, 1):
                kk = kh * 2 + b
                nb = 1 - b
                wait_data(b)
                wait_idx(nb)
                issue_data(kk + 1, nb)

                @pl.loop(0, EB)
                def _(r):
                    for j in range(C // 16):
                        sl = (r, pl.ds(j * 16, 16))
                        xrows[b][sl] = xrows[b][sl] * wfv[b][sl]

                @pl.when(kk < nchunks)
                def _():
                    pltpu.sync_copy(xrows[b], shared.at[didx[b]],
                                    add=True)

                issue_idx(kk + 2, b)

        wait_idx((maxn + 1) % 2)
        wait_data(maxn % 2)
        plsc.subcore_barrier()
        pltpu.sync_copy(shared.at[pl.ds(s * rows_per_sub, rows_per_sub)],
                        out_hbm.at[c, pl.ds(s * rows_per_sub, rows_per_sub)])

    return k(x, wf, src, dst, zeros)


def _final_body(*refs):
    agg_refs = refs[:NSPLIT]
    (z_ref, b_ref, th_ref, w2_ref, b2_ref,
     row1_ref, rob1_ref, row2_ref, rob2_ref, out_ref, gacc) = refs[NSPLIT:]
    i = pl.program_id(0)

    @pl.when(i == 0)
    def _():
        gacc[...] = jnp.zeros_like(gacc)

    agg = agg_refs[0][0] + agg_refs[0][1]
    for ar in agg_refs[1:]:
        agg = agg + ar[0] + ar[1]
    t = _ssp(jnp.dot(agg, w2_ref[...],
                     preferred_element_type=jnp.float32) + b2_ref[...])
    zb = z_ref[0, 0, :]
    oh = (zb[:, None] == lax.broadcasted_iota(jnp.int32, (NB, NE), 1)
          ).astype(jnp.float32)
    h = jnp.dot(oh, th_ref[...], preferred_element_type=jnp.float32)
    hn = h + t
    bb = b_ref[0, 0, :]
    sm = (lax.broadcasted_iota(jnp.int32, (NG, NB), 0) == bb[None, :]
          ).astype(jnp.float32)
    gacc[...] += jnp.dot(sm, hn, preferred_element_type=jnp.float32)

    r = _ssp(jnp.dot(gacc[...], row1_ref[...],
                     preferred_element_type=jnp.float32) + rob1_ref[...])
    fin = _ssp(jnp.sum(r * row2_ref[...], axis=1, keepdims=True)
               + rob2_ref[...])
    out_ref[...] = jnp.broadcast_to(fin, (NG, C))


def _final(aggs, z3, batch3, th, lin2_W, lin2_b, ro_W1, ro_b1, ro_w2row,
           ro_b2s):
    return pl.pallas_call(
        _final_body,
        grid=(NBLK,),
        in_specs=[
            pl.BlockSpec((NC, NB, C), lambda i: (0, i, 0))
            for _ in range(NSPLIT)
        ] + [
            pl.BlockSpec((1, 1, NB), lambda i: (i, 0, 0)),
            pl.BlockSpec((1, 1, NB), lambda i: (i, 0, 0)),
            pl.BlockSpec((NE, C), lambda i: (0, 0)),
            pl.BlockSpec((C, C), lambda i: (0, 0)),
            pl.BlockSpec((1, C), lambda i: (0, 0)),
            pl.BlockSpec((C, C), lambda i: (0, 0)),
            pl.BlockSpec((1, C), lambda i: (0, 0)),
            pl.BlockSpec((1, C), lambda i: (0, 0)),
            pl.BlockSpec((1, 1), lambda i: (0, 0)),
        ],
        out_specs=pl.BlockSpec((NG, C), lambda i: (0, 0)),
        out_shape=jax.ShapeDtypeStruct((NG, C), jnp.float32),
        scratch_shapes=[pltpu.VMEM((NG, C), jnp.float32)],
    )(*aggs, z3, batch3, th, lin2_W, lin2_b, ro_W1, ro_b1, ro_w2row, ro_b2s)


def kernel(z, edge_index, edge_weight, edge_attr, batch,
           emb_table, lin_emb_W, lin_emb_b,
           filt_W1, filt_b1, filt_W2, filt_b2,
           lin1_W, lin2_W, lin2_b,
           ro_W1, ro_b1, ro_W2, ro_b2):
    src = edge_index[0]
    dst = edge_index[1]
    z3 = z.reshape(NBLK, 1, NB)
    batch3 = batch.reshape(NBLK, 1, NB)
    ea_t = edge_attr.T
    ew3 = edge_weight.reshape(EBLK, _EROW, 128)
    zeros = jnp.zeros((NPAD, C), jnp.float32)

    x, th = _node_feats(z3, emb_table, lin_emb_W,
                        lin_emb_b.reshape(1, C), lin1_W)
    aggs = []
    for s in range(NSPLIT):
        wf = _filter(ea_t, ew3, filt_W1, filt_b1.reshape(1, C),
                     filt_W2, filt_b2.reshape(1, C), s)
        aggs.append(_sc_gather_mul_scatter(x, wf, src, dst, zeros, s))
    out128 = _final(aggs, z3, batch3, th, lin2_W, lin2_b.reshape(1, C),
                    ro_W1, ro_b1.reshape(1, C), ro_W2.reshape(1, C),
                    ro_b2.reshape(1, 1))
    return out128[:, :1]
